# baseline (speedup 1.0000x reference)
import functools

import jax
import jax.numpy as jnp
from jax import lax
from jax.experimental import pallas as pl
from jax.experimental.pallas import tpu as pltpu
from jax.experimental.pallas import tpu_sc as plsc

_NUM_CORES = 2
_NUM_SUBCORES = 16
_NUM_WORKERS = _NUM_CORES * _NUM_SUBCORES


@functools.cache
def _make_gather_kernel(B, D, dtype):
    b_per_w = B // _NUM_WORKERS
    mesh = plsc.VectorSubcoreMesh(core_axis_name="c", subcore_axis_name="s")
    out = jax.ShapeDtypeStruct((B, D), dtype)

    @functools.partial(
        pl.kernel,
        mesh=mesh,
        out_type=(out, out),
        compiler_params=pltpu.CompilerParams(use_tc_tiling_on_sc=False),
        scratch_types=[
            pltpu.VMEM((b_per_w,), jnp.int32),
            pltpu.VMEM((b_per_w,), jnp.int32),
            pltpu.VMEM((b_per_w, D), dtype),
            pltpu.VMEM((b_per_w, D), dtype),
            pltpu.SemaphoreType.DMA,
            pltpu.SemaphoreType.DMA,
        ],
    )
    def k(user_tab, item_tab, u_idx, i_idx, u_out, i_out,
          uidx_v, iidx_v, urows_v, irows_v, usem, isem):
        wid = lax.axis_index("s") * _NUM_CORES + lax.axis_index("c")
        base = wid * b_per_w
        pltpu.sync_copy(u_idx.at[pl.ds(base, b_per_w)], uidx_v)
        pltpu.sync_copy(i_idx.at[pl.ds(base, b_per_w)], iidx_v)
        ucp = pltpu.async_copy(user_tab.at[uidx_v], urows_v, usem)
        icp = pltpu.async_copy(item_tab.at[iidx_v], irows_v, isem)
        ucp.wait()
        pltpu.sync_copy(urows_v, u_out.at[pl.ds(base, b_per_w)])
        icp.wait()
        pltpu.sync_copy(irows_v, i_out.at[pl.ds(base, b_per_w)])

    return k


@jax.jit
def kernel(user, item, user_table, item_table):
    B = user.shape[0]
    D = user_table.shape[1]
    k = _make_gather_kernel(B, D, user_table.dtype)
    return k(user_table, item_table,
             user.astype(jnp.int32), item.astype(jnp.int32))


# speedup vs baseline: 1.4967x; 1.4967x over previous
import functools

import jax
import jax.numpy as jnp
from jax import lax
from jax.experimental import pallas as pl
from jax.experimental.pallas import tpu as pltpu
from jax.experimental.pallas import tpu_sc as plsc

_NUM_CORES = 2
_NUM_SUBCORES = 16
_NUM_WORKERS = _NUM_CORES * _NUM_SUBCORES


@functools.cache
def _make_gather_kernel(B, D, dtype):
    b_per_w = B // _NUM_WORKERS
    ch = b_per_w // 2
    mesh = plsc.VectorSubcoreMesh(core_axis_name="c", subcore_axis_name="s")
    out = jax.ShapeDtypeStruct((B, D), dtype)

    @functools.partial(
        pl.kernel,
        mesh=mesh,
        out_type=(out, out),
        scratch_types=[
            pltpu.VMEM((b_per_w,), jnp.int32),
            pltpu.VMEM((b_per_w,), jnp.int32),
            pltpu.VMEM((ch, D), dtype),
            pltpu.VMEM((ch, D), dtype),
            pltpu.SemaphoreType.DMA,
            pltpu.SemaphoreType.DMA,
        ],
    )
    def k(user_tab, item_tab, u_idx, i_idx, u_out, i_out,
          uidx_v, iidx_v, urows_v, irows_v, usem, isem):
        wid = lax.axis_index("s") * _NUM_CORES + lax.axis_index("c")
        base = wid * b_per_w
        pltpu.sync_copy(u_idx.at[pl.ds(base, b_per_w)], uidx_v)
        pltpu.sync_copy(i_idx.at[pl.ds(base, b_per_w)], iidx_v)

        @pl.loop(0, 2)
        def _(c):
            cbase = c * ch

            @plsc.parallel_loop(0, ch, step=16, unroll=2)
            def _(j):
                uv = uidx_v[pl.ds(cbase + j, 16)]
                iv = iidx_v[pl.ds(cbase + j, 16)]
                for t in range(16):
                    pltpu.async_copy(
                        user_tab.at[pl.ds(uv[t], 1), :],
                        urows_v.at[pl.ds(j + t, 1), :], usem)
                    pltpu.async_copy(
                        item_tab.at[pl.ds(iv[t], 1), :],
                        irows_v.at[pl.ds(j + t, 1), :], isem)

            pltpu.make_async_copy(
                user_tab.at[pl.ds(0, ch), :], urows_v, usem).wait()
            pltpu.make_async_copy(
                item_tab.at[pl.ds(0, ch), :], irows_v, isem).wait()

            pltpu.sync_copy(urows_v, u_out.at[pl.ds(base + cbase, ch)])
            pltpu.sync_copy(irows_v, i_out.at[pl.ds(base + cbase, ch)])

    return k


@jax.jit
def kernel(user, item, user_table, item_table):
    B = user.shape[0]
    D = user_table.shape[1]
    k = _make_gather_kernel(B, D, user_table.dtype)
    return k(user_table, item_table,
             user.astype(jnp.int32), item.astype(jnp.int32))
